# baseline (speedup 1.0000x reference)
import functools

import jax
import jax.numpy as jnp
from jax import lax
from jax.experimental import pallas as pl
from jax.experimental.pallas import tpu as pltpu
from jax.experimental.pallas import tpu_sc as plsc

V = 100001
D = 64
B = 4096
L = 50
NC, NS = 2, 16
NW = NC * NS
BPW = B // NW
EPC = 2
NCHUNK = BPW // EPC
ROWS = EPC * L


def _sc_dots(l_cxt, r_cxt, l_lbl, r_lbl, cxt_table, lbl_table):
    mesh = plsc.VectorSubcoreMesh(core_axis_name="c", subcore_axis_name="s")

    @functools.partial(
        pl.kernel,
        out_type=jax.ShapeDtypeStruct((2, NW, BPW), jnp.float32),
        mesh=mesh,
        scratch_types=[
            pltpu.VMEM((2, BPW, 56), jnp.int32),
            pltpu.VMEM((2, BPW), jnp.int32),
            pltpu.VMEM((2, BPW, D), jnp.float32),
            pltpu.VMEM((8, 56, D), jnp.float32),
            pltpu.VMEM((2, BPW), jnp.float32),
            [pltpu.SemaphoreType.DMA] * 8,
            pltpu.SemaphoreType.DMA,
        ],
        compiler_params=pltpu.CompilerParams(use_tc_tiling_on_sc=False),
    )
    def kern(l_cxt_hbm, r_cxt_hbm, l_lbl_hbm, r_lbl_hbm,
             cxt_tab_hbm, lbl_tab_hbm, out_hbm,
             idx_v, lidx_v, lrows_v, buf_v, out_v, sems, sem_l):
        wid = lax.axis_index("s") * NC + lax.axis_index("c")
        base = wid * BPW

        pltpu.sync_copy(l_cxt_hbm.at[pl.ds(base, BPW), pl.ds(0, 56)], idx_v.at[0])
        pltpu.sync_copy(r_cxt_hbm.at[pl.ds(base, BPW), pl.ds(0, 56)], idx_v.at[1])
        pltpu.sync_copy(l_lbl_hbm.at[pl.ds(base, BPW)], lidx_v.at[0])
        pltpu.sync_copy(r_lbl_hbm.at[pl.ds(base, BPW)], lidx_v.at[1])
        pltpu.async_copy(lbl_tab_hbm.at[lidx_v.at[0]], lrows_v.at[0], sem_l).wait()
        pltpu.async_copy(lbl_tab_hbm.at[lidx_v.at[1]], lrows_v.at[1], sem_l).wait()

        lanes = lax.iota(jnp.int32, 16)
        for s in range(2):
            for b in range(8):
                pltpu.async_copy(
                    cxt_tab_hbm.at[idx_v.at[s, b]], buf_v.at[b], sems[b])

            def group16(g, _, s=s):
                dvec = jnp.zeros((16,), jnp.float32)
                for b16 in range(16):
                    bb = 16 * g + b16
                    slot = b16 % 8
                    pltpu.make_async_copy(
                        cxt_tab_hbm.at[idx_v.at[s, slot]], buf_v.at[slot],
                        sems[slot]).wait()
                    lbl = [lrows_v[s, bb, pl.ds(16 * c, 16)]
                           for c in range(4)]

                    def row_acc(l, acc, slot=slot, lbl=lbl):
                        return tuple(
                            acc[c] + buf_v[slot, l, pl.ds(16 * c, 16)]
                            * lbl[c]
                            for c in range(4))

                    z = jnp.zeros((16,), jnp.float32)
                    a = lax.fori_loop(0, L, row_acc, (z, z, z, z),
                                      unroll=10)
                    tot = (a[0] + a[1]) + (a[2] + a[3])
                    for sh in (8, 4, 2, 1):
                        tot = tot + tot.at[lanes ^ sh].get(
                            mode="promise_in_bounds")
                    dvec = jnp.where(lanes == b16, tot, dvec)
                    @pl.when(bb + 8 < BPW)
                    def _(slot=slot, bb=bb, s=s):
                        pltpu.async_copy(
                            cxt_tab_hbm.at[idx_v.at[s, bb + 8]],
                            buf_v.at[slot], sems[slot])
                out_v[s, pl.ds(g * 16, 16)] = dvec
                return 0

            lax.fori_loop(0, BPW // 16, group16, 0)

        pltpu.sync_copy(out_v.at[0], out_hbm.at[0, wid])
        pltpu.sync_copy(out_v.at[1], out_hbm.at[1, wid])

    return kern(l_cxt, r_cxt, l_lbl, r_lbl, cxt_table, lbl_table)


def _tc_loss(dots):
    def body(d_ref, o_ref):
        d = d_ref[...] * (1.0 / L)
        x = jnp.where(jnp.arange(2)[:, None] == 0, d, -d)
        sp = jnp.maximum(x, 0.0) + jnp.log1p(jnp.exp(-jnp.abs(x)))
        o_ref[0, 0] = jnp.sum(sp)

    out = pl.pallas_call(
        body,
        out_shape=jax.ShapeDtypeStruct((1, 1), jnp.float32),
        out_specs=pl.BlockSpec(memory_space=pltpu.SMEM),
    )(dots)
    return out[0, 0]


def kernel(l_cxt, r_cxt, l_lbl, r_lbl, cxt_table, lbl_table):
    dots = _sc_dots(
        jnp.pad(l_cxt.astype(jnp.int32), ((0, 0), (0, 128 - L))),
        jnp.pad(r_cxt.astype(jnp.int32), ((0, 0), (0, 128 - L))),
        (l_lbl - V).astype(jnp.int32), (r_lbl - V).astype(jnp.int32),
        cxt_table, lbl_table)
    return _tc_loss(dots.reshape(2, B))


# speedup vs baseline: 6.8456x; 6.8456x over previous
import functools

import jax
import jax.numpy as jnp
from jax import lax
from jax.experimental import pallas as pl
from jax.experimental.pallas import tpu as pltpu
from jax.experimental.pallas import tpu_sc as plsc

V = 100001
D = 64
B = 4096
L = 50
NC, NS = 2, 16
NW = NC * NS
BPW = B // NW
VCH = 16384
NBLK = 7
PV = NBLK * VCH
Q = VCH // 4
MAXBLK = (V - 1) // Q
DW = D // 2


def _tc_flatten(tabT):
    def body(i0_ref, i1_ref, i2_ref, i3_ref, o_ref):
        ident = (lax.broadcasted_iota(jnp.int32, (D, D), 0)
                 == lax.broadcasted_iota(jnp.int32, (D, D), 1)
                 ).astype(jnp.float32)
        dn = (((0,), (0,)), ((), ()))

        def pack(r):
            t = lax.dot_general(r[...], ident, dn,
                                preferred_element_type=jnp.float32)
            u = lax.bitcast_convert_type(t, jnp.int32)
            r16 = (u + 0x7FFF + ((u >> 16) & 1)) >> 16
            lo = r16[:, :DW] & 0xFFFF
            hi = r16[:, DW:] << 16
            return lo | hi

        o_ref[...] = jnp.concatenate(
            [pack(i0_ref), pack(i1_ref), pack(i2_ref), pack(i3_ref)], axis=1)

    return pl.pallas_call(
        body,
        grid=(NBLK,),
        in_specs=[pl.BlockSpec((D, Q),
                               lambda g, q=q: (0, jnp.minimum(4 * g + q,
                                                              MAXBLK)))
                  for q in range(4)],
        out_specs=pl.BlockSpec((Q, 128), lambda g: (g, 0)),
        out_shape=jax.ShapeDtypeStruct((PV // 4, 128), jnp.int32),
    )(tabT, tabT, tabT, tabT)


def _remap(v):
    return (v & ~(VCH - 1)) | ((v & (Q - 1)) << 2) | ((v // Q) & 3)


def _sc_dots(l_cxt, r_cxt, l_lbl, r_lbl, cxt_table, lbl_table):
    mesh = plsc.VectorSubcoreMesh(core_axis_name="c", subcore_axis_name="s")

    @functools.partial(
        pl.kernel,
        out_type=jax.ShapeDtypeStruct((2, NW, BPW), jnp.float32),
        mesh=mesh,
        scratch_types=[
            pltpu.VMEM((2, BPW, L), jnp.int32),
            pltpu.VMEM((2, BPW), jnp.int32),
            pltpu.VMEM((2, BPW, DW), jnp.int32),
            pltpu.VMEM((8, L, DW), jnp.int32),
            pltpu.VMEM((2, BPW), jnp.float32),
            [pltpu.SemaphoreType.DMA] * 8,
            pltpu.SemaphoreType.DMA,
        ],
        compiler_params=pltpu.CompilerParams(use_tc_tiling_on_sc=False,
                                             needs_layout_passes=False),
    )
    def kern(l_cxt_hbm, r_cxt_hbm, l_lbl_hbm, r_lbl_hbm,
             cxt_tab_hbm, lbl_tab_hbm, out_hbm,
             idx_v, lidx_v, lrows_v, buf_v, out_v, sems, sem_l):
        wid = lax.axis_index("s") * NC + lax.axis_index("c")
        base = wid * BPW

        pltpu.sync_copy(l_cxt_hbm.at[pl.ds(base, BPW)], idx_v.at[0])
        pltpu.sync_copy(r_cxt_hbm.at[pl.ds(base, BPW)], idx_v.at[1])
        pltpu.sync_copy(l_lbl_hbm.at[pl.ds(base, BPW)], lidx_v.at[0])
        pltpu.sync_copy(r_lbl_hbm.at[pl.ds(base, BPW)], lidx_v.at[1])
        pltpu.async_copy(lbl_tab_hbm.at[lidx_v.at[0]], lrows_v.at[0], sem_l).wait()
        pltpu.async_copy(lbl_tab_hbm.at[lidx_v.at[1]], lrows_v.at[1], sem_l).wait()

        lanes = lax.iota(jnp.int32, 16)
        for s in range(2):
            for b in range(8):
                pltpu.async_copy(
                    cxt_tab_hbm.at[idx_v.at[s, b]], buf_v.at[b], sems[b])

            def group16(g, _, s=s):
                dvec = jnp.zeros((16,), jnp.float32)
                for b16 in range(16):
                    bb = 16 * g + b16
                    slot = b16 % 8
                    pltpu.make_async_copy(
                        cxt_tab_hbm.at[idx_v.at[s, slot]], buf_v.at[slot],
                        sems[slot]).wait()
                    himask = jnp.int32(-65536)

                    def unpk(w):
                        return (plsc.bitcast(w << 16, jnp.float32),
                                plsc.bitcast(w & himask, jnp.float32))

                    lbl = []
                    for c in range(2):
                        lbl += list(unpk(lrows_v[s, bb, pl.ds(16 * c, 16)]))

                    def row_acc(l, acc, slot=slot, lbl=lbl):
                        out = []
                        for c in range(2):
                            xe, xo = unpk(buf_v[slot, l, pl.ds(16 * c, 16)])
                            out.append(acc[2 * c] + xe * lbl[2 * c])
                            out.append(acc[2 * c + 1] + xo * lbl[2 * c + 1])
                        return tuple(out)

                    z = jnp.zeros((16,), jnp.float32)
                    a = lax.fori_loop(0, L, row_acc, (z, z, z, z),
                                      unroll=10)
                    tot = (a[0] + a[1]) + (a[2] + a[3])
                    for sh in (8, 4, 2, 1):
                        tot = tot + tot.at[lanes ^ sh].get(
                            mode="promise_in_bounds")
                    dvec = jnp.where(lanes == b16, tot, dvec)
                    @pl.when(bb + 8 < BPW)
                    def _(slot=slot, bb=bb, s=s):
                        pltpu.async_copy(
                            cxt_tab_hbm.at[idx_v.at[s, bb + 8]],
                            buf_v.at[slot], sems[slot])
                out_v[s, pl.ds(g * 16, 16)] = dvec
                return 0

            lax.fori_loop(0, BPW // 16, group16, 0)

        pltpu.sync_copy(out_v.at[0], out_hbm.at[0, wid])
        pltpu.sync_copy(out_v.at[1], out_hbm.at[1, wid])

    return kern(l_cxt, r_cxt, l_lbl, r_lbl, cxt_table, lbl_table)


def _tc_loss(dots):
    def body(d_ref, o_ref):
        d = d_ref[...] * (1.0 / L)
        x = jnp.where(jnp.arange(2)[:, None] == 0, d, -d)
        sp = jnp.maximum(x, 0.0) + jnp.log1p(jnp.exp(-jnp.abs(x)))
        o_ref[0, 0] = jnp.sum(sp)

    out = pl.pallas_call(
        body,
        out_shape=jax.ShapeDtypeStruct((1, 1), jnp.float32),
        out_specs=pl.BlockSpec(memory_space=pltpu.SMEM),
    )(dots)
    return out[0, 0]


def kernel(l_cxt, r_cxt, l_lbl, r_lbl, cxt_table, lbl_table):
    tab_c = _tc_flatten(cxt_table.T).reshape(PV, DW)
    tab_l = _tc_flatten(lbl_table.T).reshape(PV, DW)
    dots = _sc_dots(
        _remap(l_cxt.astype(jnp.int32)), _remap(r_cxt.astype(jnp.int32)),
        _remap((l_lbl - V).astype(jnp.int32)),
        _remap((r_lbl - V).astype(jnp.int32)),
        tab_c, tab_l)
    return _tc_loss(dots.reshape(2, B))


# speedup vs baseline: 8.1445x; 1.1897x over previous
import functools

import jax
import jax.numpy as jnp
from jax import lax
from jax.experimental import pallas as pl
from jax.experimental.pallas import tpu as pltpu
from jax.experimental.pallas import tpu_sc as plsc

V = 100001
D = 64
B = 4096
L = 50
NC, NS = 2, 16
NW = NC * NS
BPW = B // NW
VCH = 16384
NBLK = 7
PV = NBLK * VCH
HALF = VCH // 2
MAXBLK = (V - 1) // HALF


def _tc_flatten(tabT):
    def body(i1_ref, i2_ref, o_ref):
        ident = (lax.broadcasted_iota(jnp.int32, (D, D), 0)
                 == lax.broadcasted_iota(jnp.int32, (D, D), 1)
                 ).astype(jnp.float32)
        dn = (((0,), (0,)), ((), ()))
        a = lax.dot_general(i1_ref[...], ident, dn,
                            preferred_element_type=jnp.float32)
        b = lax.dot_general(i2_ref[...], ident, dn,
                            preferred_element_type=jnp.float32)
        o_ref[...] = jnp.concatenate([a, b], axis=1)

    return pl.pallas_call(
        body,
        grid=(NBLK,),
        in_specs=[pl.BlockSpec((D, HALF),
                               lambda g: (0, jnp.minimum(2 * g, MAXBLK))),
                  pl.BlockSpec((D, HALF),
                               lambda g: (0, jnp.minimum(2 * g + 1, MAXBLK)))],
        out_specs=pl.BlockSpec((HALF, 128), lambda g: (g, 0)),
        out_shape=jax.ShapeDtypeStruct((PV // 2, 128), jnp.float32),
    )(tabT, tabT)


def _remap(v):
    return (v & ~(VCH - 1)) | ((v & (HALF - 1)) << 1) | ((v // HALF) & 1)


def _sc_dots(l_cxt, r_cxt, l_lbl, r_lbl, cxt_table, lbl_table):
    mesh = plsc.VectorSubcoreMesh(core_axis_name="c", subcore_axis_name="s")

    @functools.partial(
        pl.kernel,
        out_type=jax.ShapeDtypeStruct((2, NW, BPW), jnp.float32),
        mesh=mesh,
        scratch_types=[
            pltpu.VMEM((2, BPW, L), jnp.int32),
            pltpu.VMEM((2, BPW), jnp.int32),
            pltpu.VMEM((2, BPW, D), jnp.float32),
            pltpu.VMEM((8, L, D), jnp.float32),
            pltpu.VMEM((2, BPW), jnp.float32),
            [pltpu.SemaphoreType.DMA] * 8,
            pltpu.SemaphoreType.DMA,
        ],
        compiler_params=pltpu.CompilerParams(use_tc_tiling_on_sc=False),
    )
    def kern(l_cxt_hbm, r_cxt_hbm, l_lbl_hbm, r_lbl_hbm,
             cxt_tab_hbm, lbl_tab_hbm, out_hbm,
             idx_v, lidx_v, lrows_v, buf_v, out_v, sems, sem_l):
        wid = lax.axis_index("s") * NC + lax.axis_index("c")
        base = wid * BPW

        pltpu.sync_copy(l_cxt_hbm.at[pl.ds(base, BPW)], idx_v.at[0])
        pltpu.sync_copy(r_cxt_hbm.at[pl.ds(base, BPW)], idx_v.at[1])
        pltpu.sync_copy(l_lbl_hbm.at[pl.ds(base, BPW)], lidx_v.at[0])
        pltpu.sync_copy(r_lbl_hbm.at[pl.ds(base, BPW)], lidx_v.at[1])
        pltpu.async_copy(lbl_tab_hbm.at[lidx_v.at[0]], lrows_v.at[0], sem_l).wait()
        pltpu.async_copy(lbl_tab_hbm.at[lidx_v.at[1]], lrows_v.at[1], sem_l).wait()

        lanes = lax.iota(jnp.int32, 16)
        for s in range(2):
            for b in range(8):
                pltpu.async_copy(
                    cxt_tab_hbm.at[idx_v.at[s, b]], buf_v.at[b], sems[b])

            def group16(g, _, s=s):
                dvec = jnp.zeros((16,), jnp.float32)
                for b16 in range(16):
                    bb = 16 * g + b16
                    slot = b16 % 8
                    pltpu.make_async_copy(
                        cxt_tab_hbm.at[idx_v.at[s, slot]], buf_v.at[slot],
                        sems[slot]).wait()
                    lbl = [lrows_v[s, bb, pl.ds(16 * c, 16)]
                           for c in range(4)]

                    def row_acc(l, acc, slot=slot, lbl=lbl):
                        return tuple(
                            acc[c] + buf_v[slot, l, pl.ds(16 * c, 16)]
                            * lbl[c]
                            for c in range(4))

                    z = jnp.zeros((16,), jnp.float32)
                    a = lax.fori_loop(0, L, row_acc, (z, z, z, z),
                                      unroll=10)
                    tot = (a[0] + a[1]) + (a[2] + a[3])
                    for sh in (8, 4, 2, 1):
                        tot = tot + tot.at[lanes ^ sh].get(
                            mode="promise_in_bounds")
                    dvec = jnp.where(lanes == b16, tot, dvec)
                    @pl.when(bb + 8 < BPW)
                    def _(slot=slot, bb=bb, s=s):
                        pltpu.async_copy(
                            cxt_tab_hbm.at[idx_v.at[s, bb + 8]],
                            buf_v.at[slot], sems[slot])
                out_v[s, pl.ds(g * 16, 16)] = dvec
                return 0

            lax.fori_loop(0, BPW // 16, group16, 0)

        pltpu.sync_copy(out_v.at[0], out_hbm.at[0, wid])
        pltpu.sync_copy(out_v.at[1], out_hbm.at[1, wid])

    return kern(l_cxt, r_cxt, l_lbl, r_lbl, cxt_table, lbl_table)


def _tc_loss(dots):
    def body(d_ref, o_ref):
        d = d_ref[...] * (1.0 / L)
        x = jnp.where(jnp.arange(2)[:, None] == 0, d, -d)
        sp = jnp.maximum(x, 0.0) + jnp.log1p(jnp.exp(-jnp.abs(x)))
        o_ref[0, 0] = jnp.sum(sp)

    out = pl.pallas_call(
        body,
        out_shape=jax.ShapeDtypeStruct((1, 1), jnp.float32),
        out_specs=pl.BlockSpec(memory_space=pltpu.SMEM),
    )(dots)
    return out[0, 0]


def kernel(l_cxt, r_cxt, l_lbl, r_lbl, cxt_table, lbl_table):
    tab_c = _tc_flatten(cxt_table.T).reshape(PV, D)
    tab_l = _tc_flatten(lbl_table.T).reshape(PV, D)
    dots = _sc_dots(
        _remap(l_cxt.astype(jnp.int32)), _remap(r_cxt.astype(jnp.int32)),
        _remap((l_lbl - V).astype(jnp.int32)),
        _remap((r_lbl - V).astype(jnp.int32)),
        tab_c, tab_l)
    return _tc_loss(dots.reshape(2, B))
